# baseline (speedup 1.0000x reference)
import functools

import jax
import jax.numpy as jnp
from jax.experimental import pallas as pl
from jax.experimental.pallas import tpu as pltpu


_TR = 16


def _chmask_body(p_ref, u_ref, o_ref):
    z = p_ref[...] + (-jnp.log(-jnp.log(u_ref[...])))
    m = jnp.max(z, axis=1, keepdims=True)
    e = jnp.exp(z - m)
    o_ref[...] = e / jnp.sum(e, axis=1, keepdims=True)


def _conv_body(x_ref, spa_ref, w_ref, rc_ref, cc_ref, o_ref, sc_ref, *,
               tr, h, w, c, jt, has_fd):
    bi = pl.program_id(0)
    ji = pl.program_id(1)

    @pl.when((bi == 0) & (ji == 0))
    def _zero_scratch():
        sc_ref[...] = jnp.zeros((h + 2, w + 2, c), jnp.bfloat16)

    @pl.when(ji < jt)
    def _fill():
        sc_ref[pl.ds(1 + ji * tr, tr), 1:w + 1, :] = x_ref[0]

    @pl.when(ji > 0)
    def _compute():
        base = (jnp.maximum(ji, 1) - 1) * tr
        wallf = w_ref[...]
        wall = wallf.astype(jnp.bfloat16)
        if has_fd:
            wd_all = (wallf * cc_ref[:, 0:1]).astype(jnp.bfloat16)
            wall = jnp.concatenate([wall, wd_all], axis=-1)
        nout = 2 * c if has_fd else c
        acc_f = jnp.zeros((tr * w, nout), jnp.float32)
        window = sc_ref[pl.ds(base, tr + 2), :, :]
        cols = [window[:, dx:dx + w, :] for dx in range(3)]
        for dy in range(3):
            for dx in range(3):
                a2 = cols[dx][dy:dy + tr].reshape(tr * w, c)
                acc_f = acc_f + jnp.dot(a2, wall[dy, dx],
                                        preferred_element_type=jnp.float32)
        spa = spa_ref[0]
        if has_fd:
            f3 = acc_f[:, :c].reshape(tr, w, c)
            c_f = rc_ref[1:2, :][None]
            c_d = rc_ref[2:3, :][None]
            r = f3 * (spa * c_f) + acc_f[:, c:].reshape(tr, w, c) * (
                (1.0 - spa) * c_d)
        else:
            c0 = rc_ref[0:1, :][None]
            c1 = rc_ref[1:2, :][None]
            r = acc_f.reshape(tr, w, c) * (c0 + spa * c1)
        o_ref[0] = jnp.maximum(r, 0.0).astype(jnp.bfloat16)


def _final_body(f0_ref, f1_ref, f2_ref, f3_ref, wc_ref, b_ref, o_ref, *,
                tr, w, c):
    acc = jnp.zeros((tr * w, c), jnp.float32) + b_ref[0:1, :]
    for i, fr in enumerate((f0_ref, f1_ref, f2_ref, f3_ref)):
        a2 = fr[0].reshape(tr * w, c)
        acc = acc + jnp.dot(a2, wc_ref[i * c:(i + 1) * c, :],
                            preferred_element_type=jnp.float32)
    o_ref[0] = acc.reshape(tr, w, c)


def kernel(x0, x1, ch_mask_param, W0, W1, W2, W3, Wc, bc, u):
    b, c, h, w = x0.shape
    n_layers = 4
    tr = _TR
    jt = h // tr

    p2 = ch_mask_param.reshape(c * n_layers, 2)
    u2 = u.reshape(c * n_layers, 2)
    cm2 = pl.pallas_call(
        _chmask_body,
        out_shape=jax.ShapeDtypeStruct((c * n_layers, 2), jnp.float32),
    )(p2, u2)
    ch_mask = cm2.reshape(1, c, n_layers, 2)

    d = cm2.reshape(c, n_layers, 2)[:, :, 0]
    s = cm2.reshape(c, n_layers, 2)[:, :, 1]

    xnhwc = jnp.transpose(x0, (0, 2, 3, 1)).astype(jnp.bfloat16)
    spa = jnp.transpose(x1, (0, 2, 3, 1))
    wts = [jnp.transpose(wi, (2, 3, 1, 0)) for wi in (W0, W1, W2, W3)]
    wc2 = jnp.transpose(Wc[:, :, 0, 0], (1, 0)).astype(jnp.bfloat16)

    z6 = jnp.zeros((6, c), jnp.float32)
    z5 = jnp.zeros((5, c), jnp.float32)
    z1 = jnp.zeros((1, c), jnp.float32)

    conv_out = jax.ShapeDtypeStruct((b, h, w, c), jnp.bfloat16)

    def layer_call(has_fd):
        return pl.pallas_call(
            functools.partial(_conv_body, tr=tr, h=h, w=w, c=c, jt=jt,
                              has_fd=has_fd),
            grid=(b, jt + 1),
            in_specs=[
                pl.BlockSpec((1, tr, w, c),
                             lambda bi, ji: (bi, jnp.minimum(ji, jt - 1), 0, 0)),
                pl.BlockSpec((1, tr, w, 1),
                             lambda bi, ji: (bi, jnp.maximum(ji, 1) - 1, 0, 0)),
                pl.BlockSpec((3, 3, c, c), lambda bi, ji: (0, 0, 0, 0)),
                pl.BlockSpec((8, c), lambda bi, ji: (0, 0)),
                pl.BlockSpec((c, 8), lambda bi, ji: (0, 0)),
            ],
            out_specs=pl.BlockSpec(
                (1, tr, w, c), lambda bi, ji: (bi, jnp.maximum(ji, 1) - 1, 0, 0)),
            out_shape=conv_out,
            scratch_shapes=[pltpu.VMEM((h + 2, w + 2, c), jnp.bfloat16)],
        )

    fea = None
    outs = []
    for i in range(n_layers):
        src = xnhwc if i == 0 else fea
        if i == 0:
            rc = jnp.concatenate([d[:, 0][None], s[:, 0][None], z6], 0)
            cc = jnp.zeros((c, 8), jnp.float32)
        else:
            rc = jnp.concatenate(
                [z1, (d[:, i] + s[:, i])[None], d[:, i][None], z5], 0)
            cc = jnp.concatenate(
                [d[:, i - 1][:, None], jnp.zeros((c, 7), jnp.float32)], 1)
        fea = layer_call(i > 0)(src, spa, wts[i], rc, cc)
        outs.append(fea)

    bias = jnp.concatenate([bc[None], jnp.zeros((7, c), jnp.float32)], 0)
    res = pl.pallas_call(
        functools.partial(_final_body, tr=tr, w=w, c=c),
        grid=(b, jt),
        in_specs=[
            pl.BlockSpec((1, tr, w, c), lambda bi, ji: (bi, ji, 0, 0)),
            pl.BlockSpec((1, tr, w, c), lambda bi, ji: (bi, ji, 0, 0)),
            pl.BlockSpec((1, tr, w, c), lambda bi, ji: (bi, ji, 0, 0)),
            pl.BlockSpec((1, tr, w, c), lambda bi, ji: (bi, ji, 0, 0)),
            pl.BlockSpec((n_layers * c, c), lambda bi, ji: (0, 0)),
            pl.BlockSpec((8, c), lambda bi, ji: (0, 0)),
        ],
        out_specs=pl.BlockSpec((1, tr, w, c), lambda bi, ji: (bi, ji, 0, 0)),
        out_shape=jax.ShapeDtypeStruct((b, h, w, c), jnp.float32),
    )(outs[0], outs[1], outs[2], outs[3], wc2, bias)

    out_final = jnp.transpose(res, (0, 3, 1, 2))
    return (out_final, ch_mask)


# speedup vs baseline: 1.0288x; 1.0288x over previous
import functools

import jax
import jax.numpy as jnp
from jax.experimental import pallas as pl
from jax.experimental.pallas import tpu as pltpu


_TR = 32


def _chmask_body(p_ref, u_ref, o_ref):
    z = p_ref[...] + (-jnp.log(-jnp.log(u_ref[...])))
    m = jnp.max(z, axis=1, keepdims=True)
    e = jnp.exp(z - m)
    o_ref[...] = e / jnp.sum(e, axis=1, keepdims=True)


def _conv_body(x_ref, spa_ref, w_ref, rc_ref, cc_ref, o_ref, sc_ref, *,
               tr, h, w, c, jt, has_fd):
    bi = pl.program_id(0)
    ji = pl.program_id(1)

    @pl.when(ji == 0)
    def _zero_border_rows():
        zrow = jnp.zeros((w + 2, c), jnp.bfloat16)
        sc_ref[0, :, :] = zrow
        sc_ref[h + 1, :, :] = zrow

    @pl.when(ji < jt)
    def _fill():
        zcol = jnp.zeros((tr, 1, c), jnp.bfloat16)
        sc_ref[pl.ds(1 + ji * tr, tr), 0:1, :] = zcol
        sc_ref[pl.ds(1 + ji * tr, tr), w + 1:w + 2, :] = zcol
        sc_ref[pl.ds(1 + ji * tr, tr), 1:w + 1, :] = x_ref[0]

    @pl.when(ji > 0)
    def _compute():
        base = (jnp.maximum(ji, 1) - 1) * tr
        wallf = w_ref[...]
        wall = wallf.astype(jnp.bfloat16)
        if has_fd:
            wd_all = (wallf * cc_ref[:, 0:1]).astype(jnp.bfloat16)
        acc_f = jnp.zeros((tr * w, c), jnp.float32)
        acc_d = jnp.zeros((tr * w, c), jnp.float32)
        window = sc_ref[pl.ds(base, tr + 2), :, :]
        cols = [window[:, dx:dx + w, :] for dx in range(3)]
        for dy in range(3):
            for dx in range(3):
                a2 = cols[dx][dy:dy + tr].reshape(tr * w, c)
                acc_f = acc_f + jnp.dot(a2, wall[dy, dx],
                                        preferred_element_type=jnp.float32)
                if has_fd:
                    acc_d = acc_d + jnp.dot(
                        a2, wd_all[dy, dx], preferred_element_type=jnp.float32)
        spa = spa_ref[0]
        f3 = acc_f.reshape(tr, w, c)
        if has_fd:
            c_f = rc_ref[1:2, :][None]
            c_d = rc_ref[2:3, :][None]
            r = f3 * (spa * c_f) + acc_d.reshape(tr, w, c) * ((1.0 - spa) * c_d)
        else:
            c0 = rc_ref[0:1, :][None]
            c1 = rc_ref[1:2, :][None]
            r = f3 * (c0 + spa * c1)
        o_ref[0] = jnp.maximum(r, 0.0).astype(jnp.bfloat16)


def _final_body(f0_ref, f1_ref, f2_ref, f3_ref, wc_ref, b_ref, o_ref, *,
                tr, w, c):
    acc = jnp.zeros((tr * w, c), jnp.float32) + b_ref[0:1, :]
    for i, fr in enumerate((f0_ref, f1_ref, f2_ref, f3_ref)):
        a2 = fr[0].reshape(tr * w, c)
        acc = acc + jnp.dot(a2, wc_ref[i * c:(i + 1) * c, :],
                            preferred_element_type=jnp.float32)
    o_ref[0] = acc.reshape(tr, w, c)


def kernel(x0, x1, ch_mask_param, W0, W1, W2, W3, Wc, bc, u):
    b, c, h, w = x0.shape
    n_layers = 4
    tr = _TR
    jt = h // tr

    p2 = ch_mask_param.reshape(c * n_layers, 2)
    u2 = u.reshape(c * n_layers, 2)
    cm2 = pl.pallas_call(
        _chmask_body,
        out_shape=jax.ShapeDtypeStruct((c * n_layers, 2), jnp.float32),
    )(p2, u2)
    ch_mask = cm2.reshape(1, c, n_layers, 2)

    d = cm2.reshape(c, n_layers, 2)[:, :, 0]
    s = cm2.reshape(c, n_layers, 2)[:, :, 1]

    xnhwc = jnp.transpose(x0, (0, 2, 3, 1)).astype(jnp.bfloat16)
    spa = jnp.transpose(x1, (0, 2, 3, 1))
    wts = [jnp.transpose(wi, (2, 3, 1, 0)) for wi in (W0, W1, W2, W3)]
    wc2 = jnp.transpose(Wc[:, :, 0, 0], (1, 0)).astype(jnp.bfloat16)

    z6 = jnp.zeros((6, c), jnp.float32)
    z5 = jnp.zeros((5, c), jnp.float32)
    z1 = jnp.zeros((1, c), jnp.float32)

    conv_out = jax.ShapeDtypeStruct((b, h, w, c), jnp.bfloat16)

    def layer_call(has_fd):
        return pl.pallas_call(
            functools.partial(_conv_body, tr=tr, h=h, w=w, c=c, jt=jt,
                              has_fd=has_fd),
            grid=(b, jt + 1),
            in_specs=[
                pl.BlockSpec((1, tr, w, c),
                             lambda bi, ji: (bi, jnp.minimum(ji, jt - 1), 0, 0)),
                pl.BlockSpec((1, tr, w, 1),
                             lambda bi, ji: (bi, jnp.maximum(ji, 1) - 1, 0, 0)),
                pl.BlockSpec((3, 3, c, c), lambda bi, ji: (0, 0, 0, 0)),
                pl.BlockSpec((8, c), lambda bi, ji: (0, 0)),
                pl.BlockSpec((c, 8), lambda bi, ji: (0, 0)),
            ],
            out_specs=pl.BlockSpec(
                (1, tr, w, c), lambda bi, ji: (bi, jnp.maximum(ji, 1) - 1, 0, 0)),
            out_shape=conv_out,
            scratch_shapes=[pltpu.VMEM((h + 2, w + 2, c), jnp.bfloat16)],
            compiler_params=pltpu.CompilerParams(
                dimension_semantics=("parallel", "arbitrary")),
        )

    fea = None
    outs = []
    for i in range(n_layers):
        src = xnhwc if i == 0 else fea
        if i == 0:
            rc = jnp.concatenate([d[:, 0][None], s[:, 0][None], z6], 0)
            cc = jnp.zeros((c, 8), jnp.float32)
        else:
            rc = jnp.concatenate(
                [z1, (d[:, i] + s[:, i])[None], d[:, i][None], z5], 0)
            cc = jnp.concatenate(
                [d[:, i - 1][:, None], jnp.zeros((c, 7), jnp.float32)], 1)
        fea = layer_call(i > 0)(src, spa, wts[i], rc, cc)
        outs.append(fea)

    bias = jnp.concatenate([bc[None], jnp.zeros((7, c), jnp.float32)], 0)
    res = pl.pallas_call(
        functools.partial(_final_body, tr=tr, w=w, c=c),
        grid=(b, jt),
        in_specs=[
            pl.BlockSpec((1, tr, w, c), lambda bi, ji: (bi, ji, 0, 0)),
            pl.BlockSpec((1, tr, w, c), lambda bi, ji: (bi, ji, 0, 0)),
            pl.BlockSpec((1, tr, w, c), lambda bi, ji: (bi, ji, 0, 0)),
            pl.BlockSpec((1, tr, w, c), lambda bi, ji: (bi, ji, 0, 0)),
            pl.BlockSpec((n_layers * c, c), lambda bi, ji: (0, 0)),
            pl.BlockSpec((8, c), lambda bi, ji: (0, 0)),
        ],
        out_specs=pl.BlockSpec((1, tr, w, c), lambda bi, ji: (bi, ji, 0, 0)),
        out_shape=jax.ShapeDtypeStruct((b, h, w, c), jnp.float32),
        compiler_params=pltpu.CompilerParams(
            dimension_semantics=("parallel", "parallel")),
    )(outs[0], outs[1], outs[2], outs[3], wc2, bias)

    out_final = jnp.transpose(res, (0, 3, 1, 2))
    return (out_final, ch_mask)
